# baseline (speedup 1.0000x reference)
import jax
import jax.numpy as jnp
from jax.experimental import pallas as pl
from jax.experimental.pallas import tpu as pltpu

_ROWS = 128
_COLS = 32768
_C = 8192
_NC = _COLS // _C
_S = 128
_NS = _C // _S


def _scan_kernel(x_ref, u_ref, o_ref, carry_ref):
    i = pl.program_id(0)

    @pl.when(i == 0)
    def _():
        carry_ref[...] = jnp.zeros_like(carry_ref)

    u = u_ref[...]
    p = carry_ref[:, :1]
    for k in range(_NS):
        xk = x_ref[:, k * _S:(k + 1) * _S]
        yk = jax.lax.dot_general(
            xk.astype(jnp.bfloat16), u, (((1,), (0,)), ((), ())),
            preferred_element_type=jnp.float32)
        yo = yk + p
        o_ref[:, k * _S:(k + 1) * _S] = yo
        p = yo[:, _S - 1:_S]
    carry_ref[:, :1] = p


def kernel(x):
    u = jnp.triu(jnp.ones((_S, _S), jnp.bfloat16))
    return pl.pallas_call(
        _scan_kernel,
        grid=(_NC,),
        in_specs=[
            pl.BlockSpec((_ROWS, _C), lambda i: (0, i)),
            pl.BlockSpec((_S, _S), lambda i: (0, 0)),
        ],
        out_specs=pl.BlockSpec((_ROWS, _C), lambda i: (0, i)),
        out_shape=jax.ShapeDtypeStruct((_ROWS, _COLS), jnp.float32),
        scratch_shapes=[pltpu.VMEM((_ROWS, 8), jnp.float32)],
    )(x, u)


# speedup vs baseline: 1.1897x; 1.1897x over previous
import jax
import jax.numpy as jnp
from jax.experimental import pallas as pl
from jax.experimental.pallas import tpu as pltpu

_ROWS = 128
_COLS = 32768
_C = 8192
_NC = _COLS // _C
_S = 128
_NS = _C // _S
_G = 8


def _scan_kernel(x_ref, u_ref, o_ref, carry_ref):
    i = pl.program_id(0)

    @pl.when(i == 0)
    def _():
        carry_ref[...] = jnp.zeros_like(carry_ref)

    u = u_ref[...]

    def _group(g, p):
        base = g * (_G * _S)
        xs = [x_ref[:, pl.ds(base + k * _S, _S)] for k in range(_G)]
        offs = []
        for k in range(_G):
            offs.append(p)
            p = p + jnp.sum(xs[k], axis=1, keepdims=True)
        for k in range(_G):
            yk = jax.lax.dot_general(
                xs[k].astype(jnp.bfloat16), u, (((1,), (0,)), ((), ())),
                preferred_element_type=jnp.float32)
            o_ref[:, pl.ds(base + k * _S, _S)] = yk + offs[k]
        return p

    p0 = carry_ref[:, :1]
    carry_ref[:, :1] = jax.lax.fori_loop(0, _NS // _G, _group, p0)


def kernel(x):
    u = jnp.triu(jnp.ones((_S, _S), jnp.bfloat16))
    return pl.pallas_call(
        _scan_kernel,
        grid=(_NC,),
        in_specs=[
            pl.BlockSpec((_ROWS, _C), lambda i: (0, i)),
            pl.BlockSpec((_S, _S), lambda i: (0, 0)),
        ],
        out_specs=pl.BlockSpec((_ROWS, _C), lambda i: (0, i)),
        out_shape=jax.ShapeDtypeStruct((_ROWS, _COLS), jnp.float32),
        scratch_shapes=[pltpu.VMEM((_ROWS, 8), jnp.float32)],
    )(x, u)


# speedup vs baseline: 1.5940x; 1.3398x over previous
import jax
import jax.numpy as jnp
from jax.experimental import pallas as pl
from jax.experimental.pallas import tpu as pltpu

_ROWS = 128
_COLS = 32768
_C = 8192
_NC = _COLS // _C
_S = 256
_NS = _C // _S


def _scan_kernel(x_ref, u_ref, o_ref, carry_ref):
    i = pl.program_id(0)

    @pl.when(i == 0)
    def _():
        carry_ref[...] = jnp.zeros_like(carry_ref)

    u = u_ref[...]
    p = carry_ref[:, :1]
    for k in range(_NS):
        xk = x_ref[:, k * _S:(k + 1) * _S]
        yk = jax.lax.dot_general(
            xk.astype(jnp.bfloat16), u, (((1,), (0,)), ((), ())),
            preferred_element_type=jnp.float32)
        o_ref[:, k * _S:(k + 1) * _S] = yk + p
        p = p + jnp.sum(xk, axis=1, keepdims=True)
    carry_ref[:, :1] = p


def kernel(x):
    u = jnp.triu(jnp.ones((_S, _S), jnp.bfloat16))
    return pl.pallas_call(
        _scan_kernel,
        grid=(_NC,),
        in_specs=[
            pl.BlockSpec((_ROWS, _C), lambda i: (0, i)),
            pl.BlockSpec((_S, _S), lambda i: (0, 0)),
        ],
        out_specs=pl.BlockSpec((_ROWS, _C), lambda i: (0, i)),
        out_shape=jax.ShapeDtypeStruct((_ROWS, _COLS), jnp.float32),
        scratch_shapes=[pltpu.VMEM((_ROWS, 8), jnp.float32)],
    )(x, u)
